# Initial kernel scaffold; baseline (speedup 1.0000x reference)
import jax
import jax.numpy as jnp
from jax.experimental import pallas as pl


def kernel(x, edge_index, W1, b1, W2, b2):
    raise NotImplementedError("write your pallas kernel here")


# speedup vs baseline: 7.8407x; 7.8407x over previous
import functools

import jax
import jax.numpy as jnp
from jax import lax
from jax.experimental import pallas as pl
from jax.experimental.pallas import tpu as pltpu
from jax.experimental.pallas import tpu_sc as plsc

N = 10000
D = 128
E = 320000

NC = 2
NS = 16
NW = NC * NS

CHUNK = 128
NCH = -(-E // (NW * CHUNK))
EP = NW * NCH * CHUNK
N_ACC = 10112
RPT = N_ACC // NS

_MESH = dict(core_axis_name="c", subcore_axis_name="s", num_cores=NC,
             num_subcores=NS)

@functools.partial(
    pl.kernel,
    out_type=jax.ShapeDtypeStruct((NC, N_ACC, D), jnp.float32),
    mesh=plsc.VectorSubcoreMesh(**_MESH),
    scratch_types=[
        pltpu.VMEM((NCH, CHUNK), jnp.int32),
        pltpu.VMEM((NCH, CHUNK), jnp.int32),
        pltpu.VMEM((NCH, CHUNK), jnp.int32),
        pltpu.VMEM((CHUNK, D), jnp.float32),
        pltpu.VMEM_SHARED((N_ACC, D), jnp.float32),
        pltpu.SemaphoreType.DMA,
    ],
)
def _sc_aggregate(feat_hbm, edges_hbm, zeros_hbm, dep_hbm, acc_out,
                  packed_v, idx_row_v, idx_col_v, rows_v, acc_sh, sem):
    del dep_hbm
    c = lax.axis_index("c")
    s = lax.axis_index("s")
    w = c * NS + s

    pltpu.sync_copy(zeros_hbm.at[pl.ds(s * RPT, RPT)],
                    acc_sh.at[pl.ds(s * RPT, RPT)])
    pltpu.sync_copy(edges_hbm.at[w], packed_v)

    def unpack(j, carry):
        for k in range(CHUNK // 16):
            p = packed_v[j, pl.ds(k * 16, 16)]
            idx_row_v[j, pl.ds(k * 16, 16)] = lax.shift_right_logical(p, 16)
            idx_col_v[j, pl.ds(k * 16, 16)] = lax.bitwise_and(p, 0xFFFF)
        return carry

    lax.fori_loop(0, NCH, unpack, 0)
    plsc.subcore_barrier()

    def step(j, carry):
        pltpu.async_copy(feat_hbm.at[idx_col_v.at[j]], rows_v, sem).wait()
        pltpu.sync_copy(rows_v, acc_sh.at[idx_row_v.at[j]], add=True)
        return carry

    lax.fori_loop(0, NCH, step, 0)
    plsc.subcore_barrier()

    pltpu.sync_copy(acc_sh.at[pl.ds(s * RPT, RPT)],
                    acc_out.at[c, pl.ds(s * RPT, RPT)])


@functools.partial(
    pl.kernel,
    out_type=jax.ShapeDtypeStruct((NC, N_ACC, D), jnp.float32),
    mesh=plsc.VectorSubcoreMesh(**_MESH),
    scratch_types=[
        pltpu.VMEM((NCH, CHUNK), jnp.int32),
        pltpu.VMEM((NCH, CHUNK), jnp.int32),
        pltpu.VMEM((CHUNK, D), jnp.float32),
        pltpu.VMEM_SHARED((N_ACC, D), jnp.float32),
    ],
)
def _sc_degree(edges_hbm, zeros_hbm, deg_out,
               packed_v, idx_row_v, ones_v, deg_sh):
    c = lax.axis_index("c")
    s = lax.axis_index("s")
    w = c * NS + s

    pltpu.sync_copy(zeros_hbm.at[pl.ds(s * RPT, RPT)],
                    deg_sh.at[pl.ds(s * RPT, RPT)])
    pltpu.sync_copy(edges_hbm.at[w], packed_v)

    def unpack(j, carry):
        for k in range(CHUNK // 16):
            p = packed_v[j, pl.ds(k * 16, 16)]
            idx_row_v[j, pl.ds(k * 16, 16)] = lax.shift_right_logical(p, 16)
        return carry

    lax.fori_loop(0, NCH, unpack, 0)

    lane0 = jnp.where(lax.iota(jnp.int32, 16) == 0, 1.0, 0.0)
    zrow = jnp.zeros((16,), jnp.float32)

    def initones(i, carry):
        ones_v[i, pl.ds(0, 16)] = lane0
        for k in range(1, D // 16):
            ones_v[i, pl.ds(k * 16, 16)] = zrow
        return carry

    lax.fori_loop(0, CHUNK, initones, 0)
    plsc.subcore_barrier()

    def step(j, carry):
        pltpu.sync_copy(ones_v, deg_sh.at[idx_row_v.at[j]], add=True)
        return carry

    lax.fori_loop(0, NCH, step, 0)
    plsc.subcore_barrier()
    pltpu.sync_copy(deg_sh.at[pl.ds(s * RPT, RPT)],
                    deg_out.at[c, pl.ds(s * RPT, RPT)])


RB = N_ACC // 4


def _mm_body(x_ref, w_ref, o_ref):
    o_ref[...] = jnp.dot(x_ref[...], w_ref[...],
                         preferred_element_type=jnp.float32)


def _tc_matmul(xp, w):
    return pl.pallas_call(
        _mm_body,
        grid=(4,),
        in_specs=[pl.BlockSpec((RB, D), lambda i: (i, 0)),
                  pl.BlockSpec((D, D), lambda i: (0, 0))],
        out_specs=pl.BlockSpec((RB, D), lambda i: (i, 0)),
        out_shape=jax.ShapeDtypeStruct((N_ACC, D), jnp.float32),
    )(xp, w)


def _combine_body(a_ref, d_ref, b_ref, w_ref, o_ref):
    s = a_ref[0] + a_ref[1]
    dg = d_ref[0, :, 0:1] + d_ref[1, :, 0:1]
    h = s * (1.0 / jnp.maximum(dg, 1.0)) + b_ref[...]
    h = jnp.where(h >= 0, h, 0.2 * h)
    o_ref[...] = jnp.dot(h, w_ref[...], preferred_element_type=jnp.float32)


def _tc_combine_mm(acc, deg, b1, w2):
    return pl.pallas_call(
        _combine_body,
        grid=(4,),
        in_specs=[pl.BlockSpec((NC, RB, D), lambda i: (0, i, 0)),
                  pl.BlockSpec((NC, RB, D), lambda i: (0, i, 0)),
                  pl.BlockSpec((1, D), lambda i: (0, 0)),
                  pl.BlockSpec((D, D), lambda i: (0, 0))],
        out_specs=pl.BlockSpec((RB, D), lambda i: (i, 0)),
        out_shape=jax.ShapeDtypeStruct((N_ACC, D), jnp.float32),
    )(acc, deg, b1, w2)


RBF = N // 5


def _final_body(a_ref, d_ref, b_ref, o_ref):
    s = a_ref[0] + a_ref[1]
    dg = d_ref[0, :, 0:1] + d_ref[1, :, 0:1]
    h = s * (1.0 / jnp.maximum(dg, 1.0)) + b_ref[...]
    nrm = jnp.sqrt(jnp.sum(h * h, axis=1, keepdims=True))
    o_ref[...] = h / jnp.maximum(nrm, 1e-12)


def _tc_finalize(acc, deg, b2):
    return pl.pallas_call(
        _final_body,
        grid=(5,),
        in_specs=[pl.BlockSpec((NC, RBF, D), lambda i: (0, i, 0)),
                  pl.BlockSpec((NC, RBF, D), lambda i: (0, i, 0)),
                  pl.BlockSpec((1, D), lambda i: (0, 0))],
        out_specs=pl.BlockSpec((RBF, D), lambda i: (i, 0)),
        out_shape=jax.ShapeDtypeStruct((N, D), jnp.float32),
    )(acc, deg, b2)


def kernel(x, edge_index, W1, b1, W2, b2):
    row = edge_index[0].astype(jnp.int32)
    col = edge_index[1].astype(jnp.int32)
    pad = EP - E
    packed = jnp.concatenate([
        jnp.left_shift(row, 16) | col,
        jnp.full((pad,), N << 16, jnp.int32),
    ]).reshape(NW, NCH, CHUNK)
    xp = jnp.pad(x, ((0, N_ACC - N), (0, 0)))
    zeros = jnp.zeros((N_ACC, D), jnp.float32)

    deg = _sc_degree(packed, zeros)
    xw1 = _tc_matmul(xp, W1)
    acc1 = _sc_aggregate(xw1, packed, zeros, deg)
    h1w2 = _tc_combine_mm(acc1, deg, b1.reshape(1, D), W2)
    acc2 = _sc_aggregate(h1w2, packed, zeros, acc1)
    return _tc_finalize(acc2, deg, b2.reshape(1, D))
